# Initial kernel scaffold; baseline (speedup 1.0000x reference)
import jax
import jax.numpy as jnp
from jax.experimental import pallas as pl


def kernel(x_cdr3b, x_tra_peptide, x_trb_peptide, ei_b2a, ei_b2b, ei_a2c, ei_b2c, edge_index_a, edge_index_b, Wl_all, bl_all, Wr_all, W1, b1, g1, be1, W2, b2, g2, be2, W3a, b3a, W3b, b3b):
    raise NotImplementedError("write your pallas kernel here")


# speedup vs baseline: 1.3554x; 1.3554x over previous
import functools

import jax
import jax.numpy as jnp
from jax import lax
from jax.experimental import pallas as pl
from jax.experimental.pallas import tpu as pltpu
from jax.experimental.pallas import tpu_sc as plsc

N = 50000
D = 128
E = 500000
P = 200000
L = 3

NS = 16

TPE = 31744
E_PAD = TPE * NS
ERPC = E_PAD // 32
TRPT = TPE // 32
NB = TRPT // 4
SPN = 50176
RPT = SPN // NS

TPP = 12800
P_PAD = TPP * NS
PRPT = TPP // 128
NBD = PRPT // 2

BM = 512
BN_INV = float(1.0 / (1.0 + 1e-5) ** 0.5)


def _leaky(t):
    return jnp.where(t > 0, t, 0.01 * t)


def _sc_counts(dsts, ones_blk, zeros_blk):
    mesh = plsc.VectorSubcoreMesh(core_axis_name="c", subcore_axis_name="s")

    @functools.partial(
        pl.kernel,
        mesh=mesh,
        compiler_params=pltpu.CompilerParams(use_tc_tiling_on_sc=False),
        out_type=[jax.ShapeDtypeStruct((SPN, 16), jnp.float32)] * 4,
        scratch_types=[
            pltpu.VMEM((4, 32), jnp.int32),
            pltpu.VMEM((32, 16), jnp.float32),
            pltpu.VMEM_SHARED((SPN, 16), jnp.float32),
            pltpu.SemaphoreType.DMA,
        ],
    )
    def body(d0, d1, d2, d3, ones_h, zeros_h, o0, o1, o2, o3,
             dvec, ones_v, acc, sem):
        core = lax.axis_index("c")
        sub = lax.axis_index("s")
        pltpu.sync_copy(ones_h, ones_v)
        for t, (dref, oref) in enumerate(((d0, o0), (d1, o1),
                                          (d2, o2), (d3, o3))):
            @pl.when(core == t // 2)
            def _():
                pltpu.sync_copy(zeros_h, acc.at[pl.ds(sub * RPT, RPT)])
                plsc.subcore_barrier()

                def bbody(b, _):
                    roff = sub * TRPT + b * 4
                    pltpu.sync_copy(dref.at[pl.ds(roff, 4)], dvec)
                    descs = [pltpu.async_copy(ones_v, acc.at[dvec.at[s]],
                                              sem, add=True)
                             for s in range(4)]
                    for dsc in descs:
                        dsc.wait()
                    return 0
                lax.fori_loop(0, NB, bbody, 0)
                plsc.subcore_barrier()
                pltpu.sync_copy(acc.at[pl.ds(sub * RPT, RPT)],
                                oref.at[pl.ds(sub * RPT, RPT)])

    return body(dsts[0], dsts[1], dsts[2], dsts[3], ones_blk, zeros_blk)


def _sc_segsum(xc4, xa4, xb4, idxs, dsts, zeros_blk):
    mesh = plsc.VectorSubcoreMesh(core_axis_name="c", subcore_axis_name="s")

    @functools.partial(
        pl.kernel,
        mesh=mesh,
        compiler_params=pltpu.CompilerParams(use_tc_tiling_on_sc=False),
        out_type=[jax.ShapeDtypeStruct((4, SPN, 32), jnp.float32)] * 4,
        scratch_types=[
            pltpu.VMEM((4, 32), jnp.int32),
            pltpu.VMEM((4, 32), jnp.int32),
            pltpu.VMEM((4, 32, 32), jnp.float32),
            pltpu.VMEM_SHARED((SPN, 32), jnp.float32),
            pltpu.SemaphoreType.DMA,
            pltpu.SemaphoreType.DMA,
        ],
    )
    def body(tc4, ta4, tb4, i0, i1, i2, i3, d0, d1, d2, d3, zeros_h,
             o0, o1, o2, o3,
             ivec, dvec, rows_v, acc, gsem, ssem):
        core = lax.axis_index("c")
        sub = lax.axis_index("s")
        for t, (tab, iref, dref, oref) in enumerate(
                ((tc4, i0, d0, o0), (tc4, i1, d1, o1),
                 (ta4, i2, d2, o2), (tb4, i3, d3, o3))):
            @pl.when(core == t // 2)
            def _():
                def cbody(cc, _):
                    pltpu.sync_copy(zeros_h, acc.at[pl.ds(sub * RPT, RPT)])
                    plsc.subcore_barrier()

                    def bbody(b, _):
                        roff = cc * ERPC + sub * TRPT + b * 4
                        droff = sub * TRPT + b * 4
                        pltpu.sync_copy(iref.at[pl.ds(roff, 4)], ivec)
                        pltpu.sync_copy(dref.at[pl.ds(droff, 4)], dvec)
                        gds = [pltpu.async_copy(tab.at[ivec.at[s]],
                                                rows_v.at[s], gsem)
                               for s in range(4)]
                        for dsc in gds:
                            dsc.wait()
                        sds = [pltpu.async_copy(rows_v.at[s],
                                                acc.at[dvec.at[s]],
                                                ssem, add=True)
                               for s in range(4)]
                        for dsc in sds:
                            dsc.wait()
                        return 0
                    lax.fori_loop(0, NB, bbody, 0)
                    plsc.subcore_barrier()
                    pltpu.sync_copy(acc.at[pl.ds(sub * RPT, RPT)],
                                    oref.at[cc, pl.ds(sub * RPT, RPT)])
                    return 0
                lax.fori_loop(0, 4, cbody, 0)

    return body(xc4, xa4, xb4, idxs[0], idxs[1], idxs[2], idxs[3],
                dsts[0], dsts[1], dsts[2], dsts[3], zeros_blk)


def _sc_pair_gather(xc, xa, xb, pidx):
    mesh = plsc.VectorSubcoreMesh(core_axis_name="c", subcore_axis_name="s")

    @functools.partial(
        pl.kernel,
        mesh=mesh,
        out_type=[jax.ShapeDtypeStruct((P_PAD, 128), jnp.float32)] * 4,
        scratch_types=[
            pltpu.VMEM((2, 128), jnp.int32),
            pltpu.VMEM((2, 128, 128), jnp.float32),
            pltpu.SemaphoreType.DMA,
        ],
    )
    def body(txc, txa, txb, i0, i1, i2, i3, o0, o1, o2, o3,
             ivec, rows_v, sem):
        core = lax.axis_index("c")
        sub = lax.axis_index("s")
        for t, (tab, iref, oref) in enumerate(
                ((txc, i0, o0), (txa, i1, o1), (txc, i2, o2), (txb, i3, o3))):
            @pl.when(core == t // 2)
            def _():
                def bbody(b, _):
                    roff = sub * PRPT + b * 2
                    eoff = sub * TPP + b * 256
                    pltpu.sync_copy(iref.at[pl.ds(roff, 2)], ivec)
                    descs = [pltpu.async_copy(tab.at[ivec.at[j]],
                                              rows_v.at[j], sem)
                             for j in range(2)]
                    for dsc in descs:
                        dsc.wait()
                    for j in range(2):
                        pltpu.sync_copy(rows_v.at[j],
                                        oref.at[pl.ds(eoff + j * 128, 128)])
                    return 0
                lax.fori_loop(0, NBD, bbody, 0)

    return body(xc, xa, xb, pidx[0], pidx[1], pidx[2], pidx[3])


def _sage1_body(cnt_ref, p_ref, x_ref, wl_ref, bl_ref, wr_ref, o_ref):
    cnt = cnt_ref[...][:, 0]
    inv = 1.0 / jnp.maximum(cnt, 1.0)
    pp = p_ref[...]
    mean = jnp.concatenate([pp[k] for k in range(4)], axis=1)
    mean = mean * inv[:, None]
    out = (jnp.dot(mean, wl_ref[...], preferred_element_type=jnp.float32)
           + jnp.dot(x_ref[...], wr_ref[...],
                     preferred_element_type=jnp.float32)
           + bl_ref[...])
    o_ref[...] = _leaky(out)


def _tc_sage1(cnt, p, x, wl, bl, wr):
    return pl.pallas_call(
        _sage1_body,
        grid=(SPN // BM,),
        in_specs=[
            pl.BlockSpec((BM, 16), lambda i: (i, 0)),
            pl.BlockSpec((4, BM, 32), lambda i: (0, i, 0)),
            pl.BlockSpec((BM, 128), lambda i: (i, 0)),
            pl.BlockSpec((128, 128), lambda i: (0, 0)),
            pl.BlockSpec((1, 128), lambda i: (0, 0)),
            pl.BlockSpec((128, 128), lambda i: (0, 0)),
        ],
        out_specs=pl.BlockSpec((BM, 128), lambda i: (i, 0)),
        out_shape=jax.ShapeDtypeStruct((N, 128), jnp.float32),
    )(cnt, p, x, wl, bl, wr)


def _sage2_body(cnta_ref, pa_ref, cntb_ref, pb_ref, x_ref,
                wla_ref, wlb_ref, bl_ref, wr_ref, o_ref):
    def mean_of(cref, pref):
        inv = 1.0 / jnp.maximum(cref[...][:, 0], 1.0)
        pp = pref[...]
        m = jnp.concatenate([pp[k] for k in range(4)], axis=1)
        return m * inv[:, None]

    ma = mean_of(cnta_ref, pa_ref)
    mb = mean_of(cntb_ref, pb_ref)
    out = (jnp.dot(ma, wla_ref[...], preferred_element_type=jnp.float32)
           + jnp.dot(mb, wlb_ref[...], preferred_element_type=jnp.float32)
           + jnp.dot(x_ref[...], wr_ref[...],
                     preferred_element_type=jnp.float32)
           + bl_ref[...])
    o_ref[...] = _leaky(out)


def _tc_sage2(cnta, pa, cntb, pb, x, wla, wlb, bl, wr):
    return pl.pallas_call(
        _sage2_body,
        grid=(SPN // BM,),
        in_specs=[
            pl.BlockSpec((BM, 16), lambda i: (i, 0)),
            pl.BlockSpec((4, BM, 32), lambda i: (0, i, 0)),
            pl.BlockSpec((BM, 16), lambda i: (i, 0)),
            pl.BlockSpec((4, BM, 32), lambda i: (0, i, 0)),
            pl.BlockSpec((BM, 128), lambda i: (i, 0)),
            pl.BlockSpec((128, 128), lambda i: (0, 0)),
            pl.BlockSpec((128, 128), lambda i: (0, 0)),
            pl.BlockSpec((1, 128), lambda i: (0, 0)),
            pl.BlockSpec((128, 128), lambda i: (0, 0)),
        ],
        out_specs=pl.BlockSpec((BM, 128), lambda i: (i, 0)),
        out_shape=jax.ShapeDtypeStruct((N, 128), jnp.float32),
    )(cnta, pa, cntb, pb, x, wla, wlb, bl, wr)


def _mlp_body(gc_ref, gx_ref, w1a_ref, w1b_ref, b1_ref, g1_ref, be1_ref,
              w2_ref, b2_ref, g2_ref, be2_ref, w3_ref, b3_ref, o_ref):
    h = (jnp.dot(gc_ref[...], w1a_ref[...], preferred_element_type=jnp.float32)
         + jnp.dot(gx_ref[...], w1b_ref[...],
                   preferred_element_type=jnp.float32))
    h = jnp.maximum((h + b1_ref[...]) * (g1_ref[...] * BN_INV) + be1_ref[...],
                    0.0)
    h = jnp.dot(h, w2_ref[...], preferred_element_type=jnp.float32)
    h = jnp.maximum((h + b2_ref[...]) * (g2_ref[...] * BN_INV) + be2_ref[...],
                    0.0)
    o_ref[...] = (jnp.dot(h, w3_ref[...], preferred_element_type=jnp.float32)
                  + b3_ref[...])


def _tc_mlp(gc, gx, w1a, w1b, b1, g1, be1, w2, b2, g2, be2, w3, b3):
    full = lambda shape: pl.BlockSpec(shape, lambda i: tuple(0 for _ in shape))
    return pl.pallas_call(
        _mlp_body,
        grid=(P_PAD // BM,),
        in_specs=[
            pl.BlockSpec((BM, 128), lambda i: (i, 0)),
            pl.BlockSpec((BM, 128), lambda i: (i, 0)),
            full((128, 512)), full((128, 512)), full((1, 512)),
            full((1, 512)), full((1, 512)),
            full((512, 256)), full((1, 256)), full((1, 256)), full((1, 256)),
            full((256, 128)), full((1, 128)),
        ],
        out_specs=pl.BlockSpec((BM, 128), lambda i: (i, 0)),
        out_shape=jax.ShapeDtypeStruct((P_PAD, 128), jnp.float32),
    )(gc, gx, w1a, w1b, b1, g1, be1, w2, b2, g2, be2, w3, b3)


def kernel(x_cdr3b, x_tra_peptide, x_trb_peptide, ei_b2a, ei_b2b, ei_a2c,
           ei_b2c, edge_index_a, edge_index_b, Wl_all, bl_all, Wr_all,
           W1, b1, g1, be1, W2, b2, g2, be2, W3a, b3a, W3b, b3b):
    f32 = jnp.float32

    def prep_edges(ei):
        src = ei[0]
        dst = ei[1]
        src_p = jnp.concatenate([src, jnp.zeros((E_PAD - E,), jnp.int32)])
        dst_p = jnp.concatenate([dst, jnp.full((E_PAD - E,), N, jnp.int32)])
        idx4 = (4 * src_p[None, :]
                + jnp.arange(4, dtype=jnp.int32)[:, None])
        idx4 = idx4.reshape(4 * ERPC, 32)
        dst2 = dst_p.reshape(ERPC, 32)
        return idx4, dst2

    i_b2a, d_b2a = prep_edges(ei_b2a)
    i_b2b, d_b2b = prep_edges(ei_b2b)
    i_a2c, d_a2c = prep_edges(ei_a2c)
    i_b2c, d_b2c = prep_edges(ei_b2c)
    idxs = (i_b2a, i_b2b, i_a2c, i_b2c)
    dsts = (d_b2a, d_b2b, d_a2c, d_b2c)

    def prep_pairs(v):
        vp = jnp.concatenate([v, jnp.zeros((P_PAD - P,), jnp.int32)])
        return vp.reshape(P_PAD // 128, 128)

    pidx = (prep_pairs(edge_index_a[0]), prep_pairs(edge_index_a[1]),
            prep_pairs(edge_index_b[0]), prep_pairs(edge_index_b[1]))

    zeros32 = jnp.zeros((RPT, 32), f32)
    zeros16 = jnp.zeros((RPT, 16), f32)
    ones16 = jnp.ones((32, 16), f32)

    cnt = _sc_counts(dsts, ones16, zeros16)

    xc, xa, xb = x_cdr3b, x_tra_peptide, x_trb_peptide
    for l in range(L):
        xc4 = xc.reshape(4 * N, 32)
        xa4 = xa.reshape(4 * N, 32)
        xb4 = xb.reshape(4 * N, 32)
        p_b2a, p_b2b, p_a2c, p_b2c = _sc_segsum(xc4, xa4, xb4, idxs, dsts,
                                                zeros32)
        xa_new = _tc_sage1(cnt[0], p_b2a, xa, Wl_all[l, 0],
                           bl_all[l, 0].reshape(1, 128), Wr_all[l, 0])
        xb_new = _tc_sage1(cnt[1], p_b2b, xb, Wl_all[l, 1],
                           bl_all[l, 1].reshape(1, 128), Wr_all[l, 1])
        xc_new = _tc_sage2(cnt[2], p_a2c, cnt[3], p_b2c, xc,
                           Wl_all[l, 2], Wl_all[l, 3],
                           (bl_all[l, 2] + bl_all[l, 3]).reshape(1, 128),
                           Wr_all[l, 2] + Wr_all[l, 3])
        xc, xa, xb = xc_new, xa_new, xb_new

    g_ca, g_a, g_cb, g_b = _sc_pair_gather(xc, xa, xb, pidx)

    w3a_p = jnp.zeros((256, 128), f32).at[:, 0:1].set(W3a)
    b3a_p = jnp.zeros((1, 128), f32).at[:, 0:1].set(b3a.reshape(1, 1))
    w3b_p = jnp.zeros((256, 128), f32).at[:, 0:1].set(W3b)
    b3b_p = jnp.zeros((1, 128), f32).at[:, 0:1].set(b3b.reshape(1, 1))

    out_a = _tc_mlp(g_ca, g_a, W1[:128], W1[128:], b1.reshape(1, 512),
                    g1.reshape(1, 512), be1.reshape(1, 512), W2,
                    b2.reshape(1, 256), g2.reshape(1, 256),
                    be2.reshape(1, 256), w3a_p, b3a_p)
    out_b = _tc_mlp(g_cb, g_b, W1[:128], W1[128:], b1.reshape(1, 512),
                    g1.reshape(1, 512), be1.reshape(1, 512), W2,
                    b2.reshape(1, 256), g2.reshape(1, 256),
                    be2.reshape(1, 256), w3b_p, b3b_p)
    return (out_a[:P, 0:1], out_b[:P, 0:1])


# speedup vs baseline: 2.5213x; 1.8601x over previous
import functools

import jax
import jax.numpy as jnp
from jax import lax
from jax.experimental import pallas as pl
from jax.experimental.pallas import tpu as pltpu
from jax.experimental.pallas import tpu_sc as plsc

N = 50000
D = 128
E = 500000
P = 200000
L = 3

NS = 16

TPE = 31744
E_PAD = TPE * NS
NROW = TPE // 128
GD = 4
NBG = NROW // GD
NPAIR = NBG // 2
SPN = 50176
RPT = SPN // NS

TPP = 12800
P_PAD = TPP * NS
PRPT = TPP // 128
NBD = PRPT // 2

BM = 512
BN_INV = float(1.0 / (1.0 + 1e-5) ** 0.5)


def _leaky(t):
    return jnp.where(t > 0, t, 0.01 * t)


def _sc_counts(dsts, ones_blk, zeros_blk):
    mesh = plsc.VectorSubcoreMesh(core_axis_name="c", subcore_axis_name="s")

    @functools.partial(
        pl.kernel,
        mesh=mesh,
        compiler_params=pltpu.CompilerParams(use_tc_tiling_on_sc=False),
        out_type=[jax.ShapeDtypeStruct((SPN, 16), jnp.float32)] * 4,
        scratch_types=[
            pltpu.VMEM((NROW, 128), jnp.int32),
            pltpu.VMEM((128, 16), jnp.float32),
            pltpu.VMEM_SHARED((SPN, 16), jnp.float32),
            pltpu.SemaphoreType.DMA,
        ],
    )
    def body(d0, d1, d2, d3, ones_h, zeros_h, o0, o1, o2, o3,
             dvec, ones_v, acc, sem):
        core = lax.axis_index("c")
        sub = lax.axis_index("s")
        pltpu.sync_copy(ones_h, ones_v)
        for t, (dref, oref) in enumerate(((d0, o0), (d1, o1),
                                          (d2, o2), (d3, o3))):
            @pl.when(core == t // 2)
            def _():
                pltpu.sync_copy(zeros_h, acc.at[pl.ds(sub * RPT, RPT)])
                pltpu.sync_copy(dref.at[sub], dvec)
                plsc.subcore_barrier()

                def bbody(g, _):
                    b0 = g * GD
                    descs = [pltpu.async_copy(ones_v, acc.at[dvec.at[b0 + j]],
                                              sem, add=True)
                             for j in range(GD)]
                    for dsc in descs:
                        dsc.wait()
                    return 0
                lax.fori_loop(0, NBG, bbody, 0)
                plsc.subcore_barrier()
                pltpu.sync_copy(acc.at[pl.ds(sub * RPT, RPT)],
                                oref.at[pl.ds(sub * RPT, RPT)])

    return body(dsts[0], dsts[1], dsts[2], dsts[3], ones_blk, zeros_blk)


def _sc_segsum(xc4, xa4, xb4, idxs, dsts, zeros_blk):
    mesh = plsc.VectorSubcoreMesh(core_axis_name="c", subcore_axis_name="s")

    @functools.partial(
        pl.kernel,
        mesh=mesh,
        compiler_params=pltpu.CompilerParams(use_tc_tiling_on_sc=False),
        out_type=[jax.ShapeDtypeStruct((4, SPN, 32), jnp.float32)] * 4,
        scratch_types=[
            pltpu.VMEM((2, GD, 128), jnp.int32),
            pltpu.VMEM((2, GD, 128), jnp.int32),
            pltpu.VMEM((GD, 128, 32), jnp.float32),
            pltpu.VMEM_SHARED((SPN, 32), jnp.float32),
            pltpu.SemaphoreType.DMA,
            pltpu.SemaphoreType.DMA,
            pltpu.SemaphoreType.DMA,
        ],
    )
    def body(tc4, ta4, tb4, i0, i1, i2, i3, d0, d1, d2, d3, zeros_h,
             o0, o1, o2, o3,
             ivec, dvec, rows_v, acc, gsem, ssem, isem):
        core = lax.axis_index("c")
        sub = lax.axis_index("s")
        for t, (tab, iref, dref, oref) in enumerate(
                ((tc4, i0, d0, o0), (tc4, i1, d1, o1),
                 (ta4, i2, d2, o2), (tb4, i3, d3, o3))):
            @pl.when(core == t // 2)
            def _():
                def cbody(cc, _):
                    pltpu.sync_copy(zeros_h, acc.at[pl.ds(sub * RPT, RPT)])
                    pltpu.async_copy(iref.at[cc, sub, pl.ds(0, GD)],
                                     ivec.at[0], isem)
                    pltpu.async_copy(dref.at[sub, pl.ds(0, GD)],
                                     dvec.at[0], isem)
                    plsc.subcore_barrier()

                    def pbody(p, _):
                        for par in range(2):
                            pltpu.make_async_copy(
                                iref.at[0, 0, pl.ds(0, GD)],
                                ivec.at[par], isem).wait()
                            pltpu.make_async_copy(
                                dref.at[0, pl.ds(0, GD)],
                                dvec.at[par], isem).wait()
                            gnext = 2 * p + par + 1

                            @pl.when(gnext < NBG)
                            def _():
                                pltpu.async_copy(
                                    iref.at[cc, sub, pl.ds(gnext * GD, GD)],
                                    ivec.at[1 - par], isem)
                                pltpu.async_copy(
                                    dref.at[sub, pl.ds(gnext * GD, GD)],
                                    dvec.at[1 - par], isem)

                            gds = [pltpu.async_copy(tab.at[ivec.at[par, j]],
                                                    rows_v.at[j], gsem)
                                   for j in range(GD)]
                            sds = []
                            for j in range(GD):
                                gds[j].wait()
                                sds.append(pltpu.async_copy(
                                    rows_v.at[j], acc.at[dvec.at[par, j]],
                                    ssem, add=True))
                            for dsc in sds:
                                dsc.wait()
                        return 0
                    lax.fori_loop(0, NPAIR, pbody, 0)
                    plsc.subcore_barrier()
                    pltpu.sync_copy(acc.at[pl.ds(sub * RPT, RPT)],
                                    oref.at[cc, pl.ds(sub * RPT, RPT)])
                    return 0
                lax.fori_loop(0, 4, cbody, 0)

    return body(xc4, xa4, xb4, idxs[0], idxs[1], idxs[2], idxs[3],
                dsts[0], dsts[1], dsts[2], dsts[3], zeros_blk)


def _sc_pair_gather(xc, xa, xb, pidx):
    mesh = plsc.VectorSubcoreMesh(core_axis_name="c", subcore_axis_name="s")

    @functools.partial(
        pl.kernel,
        mesh=mesh,
        out_type=[jax.ShapeDtypeStruct((P_PAD, 128), jnp.float32)] * 4,
        scratch_types=[
            pltpu.VMEM((2, 128), jnp.int32),
            pltpu.VMEM((2, 128, 128), jnp.float32),
            pltpu.SemaphoreType.DMA,
        ],
    )
    def body(txc, txa, txb, i0, i1, i2, i3, o0, o1, o2, o3,
             ivec, rows_v, sem):
        core = lax.axis_index("c")
        sub = lax.axis_index("s")
        for t, (tab, iref, oref) in enumerate(
                ((txc, i0, o0), (txa, i1, o1), (txc, i2, o2), (txb, i3, o3))):
            @pl.when(core == t // 2)
            def _():
                def bbody(b, _):
                    roff = sub * PRPT + b * 2
                    eoff = sub * TPP + b * 256
                    pltpu.sync_copy(iref.at[pl.ds(roff, 2)], ivec)
                    descs = [pltpu.async_copy(tab.at[ivec.at[j]],
                                              rows_v.at[j], sem)
                             for j in range(2)]
                    for dsc in descs:
                        dsc.wait()
                    for j in range(2):
                        pltpu.sync_copy(rows_v.at[j],
                                        oref.at[pl.ds(eoff + j * 128, 128)])
                    return 0
                lax.fori_loop(0, NBD, bbody, 0)

    return body(xc, xa, xb, pidx[0], pidx[1], pidx[2], pidx[3])


def _sage1_body(cnt_ref, p_ref, x_ref, wl_ref, bl_ref, wr_ref, o_ref):
    cnt = cnt_ref[...][:, 0]
    inv = 1.0 / jnp.maximum(cnt, 1.0)
    pp = p_ref[...]
    mean = jnp.concatenate([pp[k] for k in range(4)], axis=1)
    mean = mean * inv[:, None]
    out = (jnp.dot(mean, wl_ref[...], preferred_element_type=jnp.float32)
           + jnp.dot(x_ref[...], wr_ref[...],
                     preferred_element_type=jnp.float32)
           + bl_ref[...])
    o_ref[...] = _leaky(out)


def _tc_sage1(cnt, p, x, wl, bl, wr):
    return pl.pallas_call(
        _sage1_body,
        grid=(SPN // BM,),
        in_specs=[
            pl.BlockSpec((BM, 16), lambda i: (i, 0)),
            pl.BlockSpec((4, BM, 32), lambda i: (0, i, 0)),
            pl.BlockSpec((BM, 128), lambda i: (i, 0)),
            pl.BlockSpec((128, 128), lambda i: (0, 0)),
            pl.BlockSpec((1, 128), lambda i: (0, 0)),
            pl.BlockSpec((128, 128), lambda i: (0, 0)),
        ],
        out_specs=pl.BlockSpec((BM, 128), lambda i: (i, 0)),
        out_shape=jax.ShapeDtypeStruct((N, 128), jnp.float32),
    )(cnt, p, x, wl, bl, wr)


def _sage2_body(cnta_ref, pa_ref, cntb_ref, pb_ref, x_ref,
                wla_ref, wlb_ref, bl_ref, wr_ref, o_ref):
    def mean_of(cref, pref):
        inv = 1.0 / jnp.maximum(cref[...][:, 0], 1.0)
        pp = pref[...]
        m = jnp.concatenate([pp[k] for k in range(4)], axis=1)
        return m * inv[:, None]

    ma = mean_of(cnta_ref, pa_ref)
    mb = mean_of(cntb_ref, pb_ref)
    out = (jnp.dot(ma, wla_ref[...], preferred_element_type=jnp.float32)
           + jnp.dot(mb, wlb_ref[...], preferred_element_type=jnp.float32)
           + jnp.dot(x_ref[...], wr_ref[...],
                     preferred_element_type=jnp.float32)
           + bl_ref[...])
    o_ref[...] = _leaky(out)


def _tc_sage2(cnta, pa, cntb, pb, x, wla, wlb, bl, wr):
    return pl.pallas_call(
        _sage2_body,
        grid=(SPN // BM,),
        in_specs=[
            pl.BlockSpec((BM, 16), lambda i: (i, 0)),
            pl.BlockSpec((4, BM, 32), lambda i: (0, i, 0)),
            pl.BlockSpec((BM, 16), lambda i: (i, 0)),
            pl.BlockSpec((4, BM, 32), lambda i: (0, i, 0)),
            pl.BlockSpec((BM, 128), lambda i: (i, 0)),
            pl.BlockSpec((128, 128), lambda i: (0, 0)),
            pl.BlockSpec((128, 128), lambda i: (0, 0)),
            pl.BlockSpec((1, 128), lambda i: (0, 0)),
            pl.BlockSpec((128, 128), lambda i: (0, 0)),
        ],
        out_specs=pl.BlockSpec((BM, 128), lambda i: (i, 0)),
        out_shape=jax.ShapeDtypeStruct((N, 128), jnp.float32),
    )(cnta, pa, cntb, pb, x, wla, wlb, bl, wr)


def _mlp_body(gc_ref, gx_ref, w1a_ref, w1b_ref, b1_ref, g1_ref, be1_ref,
              w2_ref, b2_ref, g2_ref, be2_ref, w3_ref, b3_ref, o_ref):
    h = (jnp.dot(gc_ref[...], w1a_ref[...], preferred_element_type=jnp.float32)
         + jnp.dot(gx_ref[...], w1b_ref[...],
                   preferred_element_type=jnp.float32))
    h = jnp.maximum((h + b1_ref[...]) * (g1_ref[...] * BN_INV) + be1_ref[...],
                    0.0)
    h = jnp.dot(h, w2_ref[...], preferred_element_type=jnp.float32)
    h = jnp.maximum((h + b2_ref[...]) * (g2_ref[...] * BN_INV) + be2_ref[...],
                    0.0)
    o_ref[...] = (jnp.dot(h, w3_ref[...], preferred_element_type=jnp.float32)
                  + b3_ref[...])


def _tc_mlp(gc, gx, w1a, w1b, b1, g1, be1, w2, b2, g2, be2, w3, b3):
    full = lambda shape: pl.BlockSpec(shape, lambda i: tuple(0 for _ in shape))
    return pl.pallas_call(
        _mlp_body,
        grid=(P_PAD // BM,),
        in_specs=[
            pl.BlockSpec((BM, 128), lambda i: (i, 0)),
            pl.BlockSpec((BM, 128), lambda i: (i, 0)),
            full((128, 512)), full((128, 512)), full((1, 512)),
            full((1, 512)), full((1, 512)),
            full((512, 256)), full((1, 256)), full((1, 256)), full((1, 256)),
            full((256, 128)), full((1, 128)),
        ],
        out_specs=pl.BlockSpec((BM, 128), lambda i: (i, 0)),
        out_shape=jax.ShapeDtypeStruct((P_PAD, 128), jnp.float32),
    )(gc, gx, w1a, w1b, b1, g1, be1, w2, b2, g2, be2, w3, b3)


def kernel(x_cdr3b, x_tra_peptide, x_trb_peptide, ei_b2a, ei_b2b, ei_a2c,
           ei_b2c, edge_index_a, edge_index_b, Wl_all, bl_all, Wr_all,
           W1, b1, g1, be1, W2, b2, g2, be2, W3a, b3a, W3b, b3b):
    f32 = jnp.float32

    def prep_edges(ei):
        src = ei[0]
        dst = ei[1]
        src_p = jnp.concatenate([src, jnp.zeros((E_PAD - E,), jnp.int32)])
        dst_p = jnp.concatenate([dst, jnp.full((E_PAD - E,), N, jnp.int32)])
        idx4 = (4 * src_p[None, :]
                + jnp.arange(4, dtype=jnp.int32)[:, None])
        idx4 = idx4.reshape(4, NS, NROW, 128)
        dst2 = dst_p.reshape(NS, NROW, 128)
        return idx4, dst2

    i_b2a, d_b2a = prep_edges(ei_b2a)
    i_b2b, d_b2b = prep_edges(ei_b2b)
    i_a2c, d_a2c = prep_edges(ei_a2c)
    i_b2c, d_b2c = prep_edges(ei_b2c)
    idxs = (i_b2a, i_b2b, i_a2c, i_b2c)
    dsts = (d_b2a, d_b2b, d_a2c, d_b2c)

    def prep_pairs(v):
        vp = jnp.concatenate([v, jnp.zeros((P_PAD - P,), jnp.int32)])
        return vp.reshape(P_PAD // 128, 128)

    pidx = (prep_pairs(edge_index_a[0]), prep_pairs(edge_index_a[1]),
            prep_pairs(edge_index_b[0]), prep_pairs(edge_index_b[1]))

    zeros32 = jnp.zeros((RPT, 32), f32)
    zeros16 = jnp.zeros((RPT, 16), f32)
    ones16 = jnp.ones((128, 16), f32)

    cnt = _sc_counts(dsts, ones16, zeros16)

    xc, xa, xb = x_cdr3b, x_tra_peptide, x_trb_peptide
    for l in range(L):
        xc4 = xc.reshape(4 * N, 32)
        xa4 = xa.reshape(4 * N, 32)
        xb4 = xb.reshape(4 * N, 32)
        p_b2a, p_b2b, p_a2c, p_b2c = _sc_segsum(xc4, xa4, xb4, idxs, dsts,
                                                zeros32)
        xa_new = _tc_sage1(cnt[0], p_b2a, xa, Wl_all[l, 0],
                           bl_all[l, 0].reshape(1, 128), Wr_all[l, 0])
        xb_new = _tc_sage1(cnt[1], p_b2b, xb, Wl_all[l, 1],
                           bl_all[l, 1].reshape(1, 128), Wr_all[l, 1])
        xc_new = _tc_sage2(cnt[2], p_a2c, cnt[3], p_b2c, xc,
                           Wl_all[l, 2], Wl_all[l, 3],
                           (bl_all[l, 2] + bl_all[l, 3]).reshape(1, 128),
                           Wr_all[l, 2] + Wr_all[l, 3])
        xc, xa, xb = xc_new, xa_new, xb_new

    g_ca, g_a, g_cb, g_b = _sc_pair_gather(xc, xa, xb, pidx)

    w3a_p = jnp.zeros((256, 128), f32).at[:, 0:1].set(W3a)
    b3a_p = jnp.zeros((1, 128), f32).at[:, 0:1].set(b3a.reshape(1, 1))
    w3b_p = jnp.zeros((256, 128), f32).at[:, 0:1].set(W3b)
    b3b_p = jnp.zeros((1, 128), f32).at[:, 0:1].set(b3b.reshape(1, 1))

    out_a = _tc_mlp(g_ca, g_a, W1[:128], W1[128:], b1.reshape(1, 512),
                    g1.reshape(1, 512), be1.reshape(1, 512), W2,
                    b2.reshape(1, 256), g2.reshape(1, 256),
                    be2.reshape(1, 256), w3a_p, b3a_p)
    out_b = _tc_mlp(g_cb, g_b, W1[:128], W1[128:], b1.reshape(1, 512),
                    g1.reshape(1, 512), be1.reshape(1, 512), W2,
                    b2.reshape(1, 256), g2.reshape(1, 256),
                    be2.reshape(1, 256), w3b_p, b3b_p)
    return (out_a[:P, 0:1], out_b[:P, 0:1])


# speedup vs baseline: 2.8406x; 1.1266x over previous
import functools

import jax
import jax.numpy as jnp
from jax import lax
from jax.experimental import pallas as pl
from jax.experimental.pallas import tpu as pltpu
from jax.experimental.pallas import tpu_sc as plsc

N = 50000
D = 128
E = 500000
P = 200000
L = 3

NS = 16

TPE = 31744
E_PAD = TPE * NS
NROW = TPE // 128
GD = 4
NBG = NROW // GD
NPAIR = NBG // 2
SPN = 50176
RPT = SPN // NS

TPP = 12800
P_PAD = TPP * NS
PRPT = TPP // 128
NBD = PRPT // 2

BM = 512
BN_INV = float(1.0 / (1.0 + 1e-5) ** 0.5)


def _leaky(t):
    return jnp.where(t > 0, t, 0.01 * t)


def _sc_counts(dsts, ones_blk, zeros_blk):
    mesh = plsc.VectorSubcoreMesh(core_axis_name="c", subcore_axis_name="s")

    @functools.partial(
        pl.kernel,
        mesh=mesh,
        compiler_params=pltpu.CompilerParams(use_tc_tiling_on_sc=False),
        out_type=[jax.ShapeDtypeStruct((SPN, 16), jnp.float32)] * 4,
        scratch_types=[
            pltpu.VMEM((NROW, 128), jnp.int32),
            pltpu.VMEM((128, 16), jnp.float32),
            pltpu.VMEM_SHARED((SPN, 16), jnp.float32),
            pltpu.SemaphoreType.DMA,
        ],
    )
    def body(d0, d1, d2, d3, ones_h, zeros_h, o0, o1, o2, o3,
             dvec, ones_v, acc, sem):
        core = lax.axis_index("c")
        sub = lax.axis_index("s")
        pltpu.sync_copy(ones_h, ones_v)
        for t, (dref, oref) in enumerate(((d0, o0), (d1, o1),
                                          (d2, o2), (d3, o3))):
            @pl.when(core == t // 2)
            def _():
                pltpu.sync_copy(zeros_h, acc.at[pl.ds(sub * RPT, RPT)])
                pltpu.sync_copy(dref.at[sub], dvec)
                plsc.subcore_barrier()

                def bbody(g, _):
                    b0 = g * GD
                    descs = [pltpu.async_copy(ones_v, acc.at[dvec.at[b0 + j]],
                                              sem, add=True)
                             for j in range(GD)]
                    for dsc in descs:
                        dsc.wait()
                    return 0
                lax.fori_loop(0, NBG, bbody, 0)
                plsc.subcore_barrier()
                pltpu.sync_copy(acc.at[pl.ds(sub * RPT, RPT)],
                                oref.at[pl.ds(sub * RPT, RPT)])

    return body(dsts[0], dsts[1], dsts[2], dsts[3], ones_blk, zeros_blk)


def _sc_segsum2(tab0, tab1, idxs, dsts, zeros_blk):
    mesh = plsc.VectorSubcoreMesh(core_axis_name="c", subcore_axis_name="s")

    @functools.partial(
        pl.kernel,
        mesh=mesh,
        compiler_params=pltpu.CompilerParams(use_tc_tiling_on_sc=False),
        out_type=[jax.ShapeDtypeStruct((4, SPN, 32), jnp.float32)] * 2,
        scratch_types=[
            pltpu.VMEM((2, GD, 128), jnp.int32),
            pltpu.VMEM((2, GD, 128), jnp.int32),
            pltpu.VMEM((GD, 128, 32), jnp.float32),
            pltpu.VMEM_SHARED((SPN, 32), jnp.float32),
            pltpu.SemaphoreType.DMA,
            pltpu.SemaphoreType.DMA,
            pltpu.SemaphoreType.DMA,
        ],
    )
    def body(t0, t1, i0, i1, d0, d1, zeros_h, o0, o1,
             ivec, dvec, rows_v, acc, gsem, ssem, isem):
        core = lax.axis_index("c")
        sub = lax.axis_index("s")
        for t, (tab, iref, dref, oref) in enumerate(
                ((t0, i0, d0, o0), (t1, i1, d1, o1))):
            @pl.when(core == t)
            def _():
                def cbody(cc, _):
                    pltpu.sync_copy(zeros_h, acc.at[pl.ds(sub * RPT, RPT)])
                    pltpu.async_copy(iref.at[cc, sub, pl.ds(0, GD)],
                                     ivec.at[0], isem)
                    pltpu.async_copy(dref.at[sub, pl.ds(0, GD)],
                                     dvec.at[0], isem)
                    plsc.subcore_barrier()

                    def pbody(p, _):
                        for par in range(2):
                            pltpu.make_async_copy(
                                iref.at[0, 0, pl.ds(0, GD)],
                                ivec.at[par], isem).wait()
                            pltpu.make_async_copy(
                                dref.at[0, pl.ds(0, GD)],
                                dvec.at[par], isem).wait()
                            gnext = 2 * p + par + 1

                            @pl.when(gnext < NBG)
                            def _():
                                pltpu.async_copy(
                                    iref.at[cc, sub, pl.ds(gnext * GD, GD)],
                                    ivec.at[1 - par], isem)
                                pltpu.async_copy(
                                    dref.at[sub, pl.ds(gnext * GD, GD)],
                                    dvec.at[1 - par], isem)

                            gds = [pltpu.async_copy(tab.at[ivec.at[par, j]],
                                                    rows_v.at[j], gsem)
                                   for j in range(GD)]
                            sds = []
                            for j in range(GD):
                                gds[j].wait()
                                sds.append(pltpu.async_copy(
                                    rows_v.at[j], acc.at[dvec.at[par, j]],
                                    ssem, add=True))
                            for dsc in sds:
                                dsc.wait()
                        return 0
                    lax.fori_loop(0, NPAIR, pbody, 0)
                    plsc.subcore_barrier()
                    pltpu.sync_copy(acc.at[pl.ds(sub * RPT, RPT)],
                                    oref.at[cc, pl.ds(sub * RPT, RPT)])
                    return 0
                lax.fori_loop(0, 4, cbody, 0)

    return body(tab0, tab1, idxs[0], idxs[1], dsts[0], dsts[1], zeros_blk)


def _sc_pair_gather2(tabA, tabB, iA, iB):
    mesh = plsc.VectorSubcoreMesh(core_axis_name="c", subcore_axis_name="s")

    @functools.partial(
        pl.kernel,
        mesh=mesh,
        out_type=[jax.ShapeDtypeStruct((P_PAD, 128), jnp.float32)] * 2,
        scratch_types=[
            pltpu.VMEM((2, 128), jnp.int32),
            pltpu.VMEM((2, 128, 128), jnp.float32),
            pltpu.SemaphoreType.DMA,
        ],
    )
    def body(t0, t1, i0, i1, o0, o1, ivec, rows_v, sem):
        core = lax.axis_index("c")
        sub = lax.axis_index("s")
        for t, (tab, iref, oref) in enumerate(((t0, i0, o0), (t1, i1, o1))):
            @pl.when(core == t)
            def _():
                def bbody(b, _):
                    roff = sub * PRPT + b * 2
                    eoff = sub * TPP + b * 256
                    pltpu.sync_copy(iref.at[pl.ds(roff, 2)], ivec)
                    descs = [pltpu.async_copy(tab.at[ivec.at[j]],
                                              rows_v.at[j], sem)
                             for j in range(2)]
                    for dsc in descs:
                        dsc.wait()
                    for j in range(2):
                        pltpu.sync_copy(rows_v.at[j],
                                        oref.at[pl.ds(eoff + j * 128, 128)])
                    return 0
                lax.fori_loop(0, NBD, bbody, 0)

    return body(tabA, tabB, iA, iB)


def _sage1_body(cnt_ref, p_ref, x_ref, wl_ref, bl_ref, wr_ref, o_ref):
    cnt = cnt_ref[...][:, 0]
    inv = 1.0 / jnp.maximum(cnt, 1.0)
    pp = p_ref[...]
    mean = jnp.concatenate([pp[k] for k in range(4)], axis=1)
    mean = mean * inv[:, None]
    out = (jnp.dot(mean, wl_ref[...], preferred_element_type=jnp.float32)
           + jnp.dot(x_ref[...], wr_ref[...],
                     preferred_element_type=jnp.float32)
           + bl_ref[...])
    o_ref[...] = _leaky(out)


def _tc_sage1(cnt, p, x, wl, bl, wr):
    return pl.pallas_call(
        _sage1_body,
        grid=(SPN // BM,),
        in_specs=[
            pl.BlockSpec((BM, 16), lambda i: (i, 0)),
            pl.BlockSpec((4, BM, 32), lambda i: (0, i, 0)),
            pl.BlockSpec((BM, 128), lambda i: (i, 0)),
            pl.BlockSpec((128, 128), lambda i: (0, 0)),
            pl.BlockSpec((1, 128), lambda i: (0, 0)),
            pl.BlockSpec((128, 128), lambda i: (0, 0)),
        ],
        out_specs=pl.BlockSpec((BM, 128), lambda i: (i, 0)),
        out_shape=jax.ShapeDtypeStruct((N, 128), jnp.float32),
    )(cnt, p, x, wl, bl, wr)


def _sage2_body(cnta_ref, pa_ref, cntb_ref, pb_ref, x_ref,
                wla_ref, wlb_ref, bl_ref, wr_ref, o_ref):
    def mean_of(cref, pref):
        inv = 1.0 / jnp.maximum(cref[...][:, 0], 1.0)
        pp = pref[...]
        m = jnp.concatenate([pp[k] for k in range(4)], axis=1)
        return m * inv[:, None]

    ma = mean_of(cnta_ref, pa_ref)
    mb = mean_of(cntb_ref, pb_ref)
    out = (jnp.dot(ma, wla_ref[...], preferred_element_type=jnp.float32)
           + jnp.dot(mb, wlb_ref[...], preferred_element_type=jnp.float32)
           + jnp.dot(x_ref[...], wr_ref[...],
                     preferred_element_type=jnp.float32)
           + bl_ref[...])
    o_ref[...] = _leaky(out)


def _tc_sage2(cnta, pa, cntb, pb, x, wla, wlb, bl, wr):
    return pl.pallas_call(
        _sage2_body,
        grid=(SPN // BM,),
        in_specs=[
            pl.BlockSpec((BM, 16), lambda i: (i, 0)),
            pl.BlockSpec((4, BM, 32), lambda i: (0, i, 0)),
            pl.BlockSpec((BM, 16), lambda i: (i, 0)),
            pl.BlockSpec((4, BM, 32), lambda i: (0, i, 0)),
            pl.BlockSpec((BM, 128), lambda i: (i, 0)),
            pl.BlockSpec((128, 128), lambda i: (0, 0)),
            pl.BlockSpec((128, 128), lambda i: (0, 0)),
            pl.BlockSpec((1, 128), lambda i: (0, 0)),
            pl.BlockSpec((128, 128), lambda i: (0, 0)),
        ],
        out_specs=pl.BlockSpec((BM, 128), lambda i: (i, 0)),
        out_shape=jax.ShapeDtypeStruct((N, 128), jnp.float32),
    )(cnta, pa, cntb, pb, x, wla, wlb, bl, wr)


def _mlp_body(gc_ref, gx_ref, w1a_ref, w1b_ref, b1_ref, g1_ref, be1_ref,
              w2_ref, b2_ref, g2_ref, be2_ref, w3_ref, b3_ref, o_ref):
    h = (jnp.dot(gc_ref[...], w1a_ref[...], preferred_element_type=jnp.float32)
         + jnp.dot(gx_ref[...], w1b_ref[...],
                   preferred_element_type=jnp.float32))
    h = jnp.maximum((h + b1_ref[...]) * (g1_ref[...] * BN_INV) + be1_ref[...],
                    0.0)
    h = jnp.dot(h, w2_ref[...], preferred_element_type=jnp.float32)
    h = jnp.maximum((h + b2_ref[...]) * (g2_ref[...] * BN_INV) + be2_ref[...],
                    0.0)
    o_ref[...] = (jnp.dot(h, w3_ref[...], preferred_element_type=jnp.float32)
                  + b3_ref[...])


def _tc_mlp(gc, gx, w1a, w1b, b1, g1, be1, w2, b2, g2, be2, w3, b3):
    full = lambda shape: pl.BlockSpec(shape, lambda i: tuple(0 for _ in shape))
    return pl.pallas_call(
        _mlp_body,
        grid=(P_PAD // BM,),
        in_specs=[
            pl.BlockSpec((BM, 128), lambda i: (i, 0)),
            pl.BlockSpec((BM, 128), lambda i: (i, 0)),
            full((128, 512)), full((128, 512)), full((1, 512)),
            full((1, 512)), full((1, 512)),
            full((512, 256)), full((1, 256)), full((1, 256)), full((1, 256)),
            full((256, 128)), full((1, 128)),
        ],
        out_specs=pl.BlockSpec((BM, 128), lambda i: (i, 0)),
        out_shape=jax.ShapeDtypeStruct((P_PAD, 128), jnp.float32),
    )(gc, gx, w1a, w1b, b1, g1, be1, w2, b2, g2, be2, w3, b3)


def kernel(x_cdr3b, x_tra_peptide, x_trb_peptide, ei_b2a, ei_b2b, ei_a2c,
           ei_b2c, edge_index_a, edge_index_b, Wl_all, bl_all, Wr_all,
           W1, b1, g1, be1, W2, b2, g2, be2, W3a, b3a, W3b, b3b):
    f32 = jnp.float32

    def prep_edges(ei):
        src = ei[0]
        dst = ei[1]
        src_p = jnp.concatenate([src, jnp.zeros((E_PAD - E,), jnp.int32)])
        dst_p = jnp.concatenate([dst, jnp.full((E_PAD - E,), N, jnp.int32)])
        idx4 = (4 * src_p[None, :]
                + jnp.arange(4, dtype=jnp.int32)[:, None])
        idx4 = idx4.reshape(4, NS, NROW, 128)
        dst2 = dst_p.reshape(NS, NROW, 128)
        return idx4, dst2

    i_b2a, d_b2a = prep_edges(ei_b2a)
    i_b2b, d_b2b = prep_edges(ei_b2b)
    i_a2c, d_a2c = prep_edges(ei_a2c)
    i_b2c, d_b2c = prep_edges(ei_b2c)
    idxs = (i_b2a, i_b2b, i_a2c, i_b2c)
    dsts = (d_b2a, d_b2b, d_a2c, d_b2c)

    def prep_pairs(v):
        vp = jnp.concatenate([v, jnp.zeros((P_PAD - P,), jnp.int32)])
        return vp.reshape(P_PAD // 128, 128)

    pidx = (prep_pairs(edge_index_a[0]), prep_pairs(edge_index_a[1]),
            prep_pairs(edge_index_b[0]), prep_pairs(edge_index_b[1]))

    zeros32 = jnp.zeros((RPT, 32), f32)
    zeros16 = jnp.zeros((RPT, 16), f32)
    ones16 = jnp.ones((128, 16), f32)

    cnt = _sc_counts(dsts, ones16, zeros16)

    xc, xa, xb = x_cdr3b, x_tra_peptide, x_trb_peptide
    for l in range(L):
        xc4 = xc.reshape(4 * N, 32)
        xa4 = xa.reshape(4 * N, 32)
        xb4 = xb.reshape(4 * N, 32)
        p_b2a, p_b2b = _sc_segsum2(xc4, xc4, (idxs[0], idxs[1]),
                                   (dsts[0], dsts[1]), zeros32)
        p_a2c, p_b2c = _sc_segsum2(xa4, xb4, (idxs[2], idxs[3]),
                                   (dsts[2], dsts[3]), zeros32)
        xa_new = _tc_sage1(cnt[0], p_b2a, xa, Wl_all[l, 0],
                           bl_all[l, 0].reshape(1, 128), Wr_all[l, 0])
        xb_new = _tc_sage1(cnt[1], p_b2b, xb, Wl_all[l, 1],
                           bl_all[l, 1].reshape(1, 128), Wr_all[l, 1])
        xc_new = _tc_sage2(cnt[2], p_a2c, cnt[3], p_b2c, xc,
                           Wl_all[l, 2], Wl_all[l, 3],
                           (bl_all[l, 2] + bl_all[l, 3]).reshape(1, 128),
                           Wr_all[l, 2] + Wr_all[l, 3])
        xc, xa, xb = xc_new, xa_new, xb_new

    g_ca, g_a = _sc_pair_gather2(xc, xa, pidx[0], pidx[1])
    g_cb, g_b = _sc_pair_gather2(xc, xb, pidx[2], pidx[3])

    w3a_p = jnp.zeros((256, 128), f32).at[:, 0:1].set(W3a)
    b3a_p = jnp.zeros((1, 128), f32).at[:, 0:1].set(b3a.reshape(1, 1))
    w3b_p = jnp.zeros((256, 128), f32).at[:, 0:1].set(W3b)
    b3b_p = jnp.zeros((1, 128), f32).at[:, 0:1].set(b3b.reshape(1, 1))

    out_a = _tc_mlp(g_ca, g_a, W1[:128], W1[128:], b1.reshape(1, 512),
                    g1.reshape(1, 512), be1.reshape(1, 512), W2,
                    b2.reshape(1, 256), g2.reshape(1, 256),
                    be2.reshape(1, 256), w3a_p, b3a_p)
    out_b = _tc_mlp(g_cb, g_b, W1[:128], W1[128:], b1.reshape(1, 512),
                    g1.reshape(1, 512), be1.reshape(1, 512), W2,
                    b2.reshape(1, 256), g2.reshape(1, 256),
                    be2.reshape(1, 256), w3b_p, b3b_p)
    return (out_a[:P, 0:1], out_b[:P, 0:1])


# speedup vs baseline: 2.8825x; 1.0148x over previous
import functools

import jax
import jax.numpy as jnp
from jax import lax
from jax.experimental import pallas as pl
from jax.experimental.pallas import tpu as pltpu
from jax.experimental.pallas import tpu_sc as plsc

N = 50000
D = 128
E = 500000
P = 200000
L = 3

NS = 16

TPE = 31744
E_PAD = TPE * NS
NROW = TPE // 128
GD = 4
NBG = NROW // GD
NPAIR = NBG // 2
SPN = 50176
RPT = SPN // NS

TPP = 12800
P_PAD = TPP * NS
PRPT = TPP // 128
GDP = 4

BM = 512
BN_INV = float(1.0 / (1.0 + 1e-5) ** 0.5)


def _leaky(t):
    return jnp.where(t > 0, t, 0.01 * t)


def _sc_counts(dsts, ones_blk, zeros_blk):
    mesh = plsc.VectorSubcoreMesh(core_axis_name="c", subcore_axis_name="s")

    @functools.partial(
        pl.kernel,
        mesh=mesh,
        compiler_params=pltpu.CompilerParams(use_tc_tiling_on_sc=False),
        out_type=[jax.ShapeDtypeStruct((SPN, 16), jnp.float32)] * 4,
        scratch_types=[
            pltpu.VMEM((NROW, 128), jnp.int32),
            pltpu.VMEM((128, 16), jnp.float32),
            pltpu.VMEM_SHARED((SPN, 16), jnp.float32),
            pltpu.SemaphoreType.DMA,
        ],
    )
    def body(d0, d1, d2, d3, ones_h, zeros_h, o0, o1, o2, o3,
             dvec, ones_v, acc, sem):
        core = lax.axis_index("c")
        sub = lax.axis_index("s")
        pltpu.sync_copy(ones_h, ones_v)
        for t, (dref, oref) in enumerate(((d0, o0), (d1, o1),
                                          (d2, o2), (d3, o3))):
            @pl.when(core == t // 2)
            def _():
                pltpu.sync_copy(zeros_h, acc.at[pl.ds(sub * RPT, RPT)])
                pltpu.sync_copy(dref.at[sub], dvec)
                plsc.subcore_barrier()

                def bbody(g, _):
                    b0 = g * GD
                    descs = [pltpu.async_copy(ones_v, acc.at[dvec.at[b0 + j]],
                                              sem, add=True)
                             for j in range(GD)]
                    for dsc in descs:
                        dsc.wait()
                    return 0
                lax.fori_loop(0, NBG, bbody, 0)
                plsc.subcore_barrier()
                pltpu.sync_copy(acc.at[pl.ds(sub * RPT, RPT)],
                                oref.at[pl.ds(sub * RPT, RPT)])

    return body(dsts[0], dsts[1], dsts[2], dsts[3], ones_blk, zeros_blk)


def _sc_segsum2(tab0, tab1, idxs, dsts, zeros_blk):
    mesh = plsc.VectorSubcoreMesh(core_axis_name="c", subcore_axis_name="s")

    @functools.partial(
        pl.kernel,
        mesh=mesh,
        compiler_params=pltpu.CompilerParams(use_tc_tiling_on_sc=False),
        out_type=[jax.ShapeDtypeStruct((4, SPN, 32), jnp.float32)] * 2,
        scratch_types=[
            pltpu.VMEM((2, GD, 128), jnp.int32),
            pltpu.VMEM((2, GD, 128), jnp.int32),
            pltpu.VMEM((GD, 128, 32), jnp.float32),
            pltpu.VMEM_SHARED((SPN, 32), jnp.float32),
            pltpu.SemaphoreType.DMA,
            pltpu.SemaphoreType.DMA,
            pltpu.SemaphoreType.DMA,
        ],
    )
    def body(t0, t1, i0, i1, d0, d1, zeros_h, o0, o1,
             ivec, dvec, rows_v, acc, gsem, ssem, isem):
        core = lax.axis_index("c")
        sub = lax.axis_index("s")
        for t, (tab, iref, dref, oref) in enumerate(
                ((t0, i0, d0, o0), (t1, i1, d1, o1))):
            @pl.when(core == t)
            def _():
                def cbody(cc, _):
                    pltpu.sync_copy(zeros_h, acc.at[pl.ds(sub * RPT, RPT)])
                    pltpu.async_copy(iref.at[cc, sub, pl.ds(0, GD)],
                                     ivec.at[0], isem)
                    pltpu.async_copy(dref.at[sub, pl.ds(0, GD)],
                                     dvec.at[0], isem)
                    plsc.subcore_barrier()

                    def pbody(p, _):
                        for par in range(2):
                            pltpu.make_async_copy(
                                iref.at[0, 0, pl.ds(0, GD)],
                                ivec.at[par], isem).wait()
                            pltpu.make_async_copy(
                                dref.at[0, pl.ds(0, GD)],
                                dvec.at[par], isem).wait()
                            gnext = 2 * p + par + 1

                            @pl.when(gnext < NBG)
                            def _():
                                pltpu.async_copy(
                                    iref.at[cc, sub, pl.ds(gnext * GD, GD)],
                                    ivec.at[1 - par], isem)
                                pltpu.async_copy(
                                    dref.at[sub, pl.ds(gnext * GD, GD)],
                                    dvec.at[1 - par], isem)

                            gds = [pltpu.async_copy(tab.at[ivec.at[par, j]],
                                                    rows_v.at[j], gsem)
                                   for j in range(GD)]
                            sds = []
                            for j in range(GD):
                                gds[j].wait()
                                sds.append(pltpu.async_copy(
                                    rows_v.at[j], acc.at[dvec.at[par, j]],
                                    ssem, add=True))
                            for dsc in sds:
                                dsc.wait()
                        return 0
                    lax.fori_loop(0, NPAIR, pbody, 0)
                    plsc.subcore_barrier()
                    pltpu.sync_copy(acc.at[pl.ds(sub * RPT, RPT)],
                                    oref.at[cc, pl.ds(sub * RPT, RPT)])
                    return 0
                lax.fori_loop(0, 4, cbody, 0)

    return body(tab0, tab1, idxs[0], idxs[1], dsts[0], dsts[1], zeros_blk)


def _sc_pair_gather2(tabA, tabB, iA, iB):
    mesh = plsc.VectorSubcoreMesh(core_axis_name="c", subcore_axis_name="s")

    @functools.partial(
        pl.kernel,
        mesh=mesh,
        out_type=[jax.ShapeDtypeStruct((P_PAD, 128), jnp.float32)] * 2,
        scratch_types=[
            pltpu.VMEM((PRPT, 128), jnp.int32),
            pltpu.VMEM((GDP, 128, 128), jnp.float32),
            pltpu.SemaphoreType.DMA,
            pltpu.SemaphoreType.DMA,
        ],
    )
    def body(t0, t1, i0, i1, o0, o1, ivec, rows_v, gsem, osem):
        core = lax.axis_index("c")
        sub = lax.axis_index("s")
        for t, (tab, iref, oref) in enumerate(((t0, i0, o0), (t1, i1, o1))):
            @pl.when(core == t)
            def _():
                pltpu.sync_copy(iref.at[sub], ivec)

                def bbody(g, _):
                    b0 = g * GDP
                    gds = [pltpu.async_copy(tab.at[ivec.at[b0 + j]],
                                            rows_v.at[j], gsem)
                           for j in range(GDP)]
                    wds = []
                    for j in range(GDP):
                        gds[j].wait()
                        wds.append(pltpu.async_copy(
                            rows_v.at[j],
                            oref.at[pl.ds(sub * TPP + (b0 + j) * 128, 128)],
                            osem))
                    for dsc in wds:
                        dsc.wait()
                    return 0
                lax.fori_loop(0, PRPT // GDP, bbody, 0)

    return body(tabA, tabB, iA, iB)


def _sage1_body(cnt_ref, p_ref, x_ref, wl_ref, bl_ref, wr_ref, o_ref):
    cnt = cnt_ref[...][:, 0]
    inv = 1.0 / jnp.maximum(cnt, 1.0)
    pp = p_ref[...]
    mean = jnp.concatenate([pp[k] for k in range(4)], axis=1)
    mean = mean * inv[:, None]
    out = (jnp.dot(mean, wl_ref[...], preferred_element_type=jnp.float32)
           + jnp.dot(x_ref[...], wr_ref[...],
                     preferred_element_type=jnp.float32)
           + bl_ref[...])
    o_ref[...] = _leaky(out)


def _tc_sage1(cnt, p, x, wl, bl, wr):
    return pl.pallas_call(
        _sage1_body,
        grid=(SPN // BM,),
        in_specs=[
            pl.BlockSpec((BM, 16), lambda i: (i, 0)),
            pl.BlockSpec((4, BM, 32), lambda i: (0, i, 0)),
            pl.BlockSpec((BM, 128), lambda i: (i, 0)),
            pl.BlockSpec((128, 128), lambda i: (0, 0)),
            pl.BlockSpec((1, 128), lambda i: (0, 0)),
            pl.BlockSpec((128, 128), lambda i: (0, 0)),
        ],
        out_specs=pl.BlockSpec((BM, 128), lambda i: (i, 0)),
        out_shape=jax.ShapeDtypeStruct((N, 128), jnp.float32),
    )(cnt, p, x, wl, bl, wr)


def _sage2_body(cnta_ref, pa_ref, cntb_ref, pb_ref, x_ref,
                wla_ref, wlb_ref, bl_ref, wr_ref, o_ref):
    def mean_of(cref, pref):
        inv = 1.0 / jnp.maximum(cref[...][:, 0], 1.0)
        pp = pref[...]
        m = jnp.concatenate([pp[k] for k in range(4)], axis=1)
        return m * inv[:, None]

    ma = mean_of(cnta_ref, pa_ref)
    mb = mean_of(cntb_ref, pb_ref)
    out = (jnp.dot(ma, wla_ref[...], preferred_element_type=jnp.float32)
           + jnp.dot(mb, wlb_ref[...], preferred_element_type=jnp.float32)
           + jnp.dot(x_ref[...], wr_ref[...],
                     preferred_element_type=jnp.float32)
           + bl_ref[...])
    o_ref[...] = _leaky(out)


def _tc_sage2(cnta, pa, cntb, pb, x, wla, wlb, bl, wr):
    return pl.pallas_call(
        _sage2_body,
        grid=(SPN // BM,),
        in_specs=[
            pl.BlockSpec((BM, 16), lambda i: (i, 0)),
            pl.BlockSpec((4, BM, 32), lambda i: (0, i, 0)),
            pl.BlockSpec((BM, 16), lambda i: (i, 0)),
            pl.BlockSpec((4, BM, 32), lambda i: (0, i, 0)),
            pl.BlockSpec((BM, 128), lambda i: (i, 0)),
            pl.BlockSpec((128, 128), lambda i: (0, 0)),
            pl.BlockSpec((128, 128), lambda i: (0, 0)),
            pl.BlockSpec((1, 128), lambda i: (0, 0)),
            pl.BlockSpec((128, 128), lambda i: (0, 0)),
        ],
        out_specs=pl.BlockSpec((BM, 128), lambda i: (i, 0)),
        out_shape=jax.ShapeDtypeStruct((N, 128), jnp.float32),
    )(cnta, pa, cntb, pb, x, wla, wlb, bl, wr)


def _mlp_body(gc_ref, gx_ref, w1a_ref, w1b_ref, b1_ref, g1_ref, be1_ref,
              w2_ref, b2_ref, g2_ref, be2_ref, w3_ref, b3_ref, o_ref):
    h = (jnp.dot(gc_ref[...], w1a_ref[...], preferred_element_type=jnp.float32)
         + jnp.dot(gx_ref[...], w1b_ref[...],
                   preferred_element_type=jnp.float32))
    h = jnp.maximum((h + b1_ref[...]) * (g1_ref[...] * BN_INV) + be1_ref[...],
                    0.0)
    h = jnp.dot(h, w2_ref[...], preferred_element_type=jnp.float32)
    h = jnp.maximum((h + b2_ref[...]) * (g2_ref[...] * BN_INV) + be2_ref[...],
                    0.0)
    o_ref[...] = (jnp.dot(h, w3_ref[...], preferred_element_type=jnp.float32)
                  + b3_ref[...])


def _tc_mlp(gc, gx, w1a, w1b, b1, g1, be1, w2, b2, g2, be2, w3, b3):
    full = lambda shape: pl.BlockSpec(shape, lambda i: tuple(0 for _ in shape))
    return pl.pallas_call(
        _mlp_body,
        grid=(P_PAD // BM,),
        in_specs=[
            pl.BlockSpec((BM, 128), lambda i: (i, 0)),
            pl.BlockSpec((BM, 128), lambda i: (i, 0)),
            full((128, 512)), full((128, 512)), full((1, 512)),
            full((1, 512)), full((1, 512)),
            full((512, 256)), full((1, 256)), full((1, 256)), full((1, 256)),
            full((256, 128)), full((1, 128)),
        ],
        out_specs=pl.BlockSpec((BM, 128), lambda i: (i, 0)),
        out_shape=jax.ShapeDtypeStruct((P_PAD, 128), jnp.float32),
    )(gc, gx, w1a, w1b, b1, g1, be1, w2, b2, g2, be2, w3, b3)


def kernel(x_cdr3b, x_tra_peptide, x_trb_peptide, ei_b2a, ei_b2b, ei_a2c,
           ei_b2c, edge_index_a, edge_index_b, Wl_all, bl_all, Wr_all,
           W1, b1, g1, be1, W2, b2, g2, be2, W3a, b3a, W3b, b3b):
    f32 = jnp.float32

    def prep_edges(ei):
        src = ei[0]
        dst = ei[1]
        src_p = jnp.concatenate([src, jnp.zeros((E_PAD - E,), jnp.int32)])
        dst_p = jnp.concatenate([dst, jnp.full((E_PAD - E,), N, jnp.int32)])
        idx4 = (4 * src_p[None, :]
                + jnp.arange(4, dtype=jnp.int32)[:, None])
        idx4 = idx4.reshape(4, NS, NROW, 128)
        dst2 = dst_p.reshape(NS, NROW, 128)
        return idx4, dst2

    i_b2a, d_b2a = prep_edges(ei_b2a)
    i_b2b, d_b2b = prep_edges(ei_b2b)
    i_a2c, d_a2c = prep_edges(ei_a2c)
    i_b2c, d_b2c = prep_edges(ei_b2c)
    idxs = (i_b2a, i_b2b, i_a2c, i_b2c)
    dsts = (d_b2a, d_b2b, d_a2c, d_b2c)

    def prep_pairs(v):
        vp = jnp.concatenate([v, jnp.zeros((P_PAD - P,), jnp.int32)])
        return vp.reshape(NS, PRPT, 128)

    pidx = (prep_pairs(edge_index_a[0]), prep_pairs(edge_index_a[1]),
            prep_pairs(edge_index_b[0]), prep_pairs(edge_index_b[1]))

    zeros32 = jnp.zeros((RPT, 32), f32)
    zeros16 = jnp.zeros((RPT, 16), f32)
    ones16 = jnp.ones((128, 16), f32)

    cnt = _sc_counts(dsts, ones16, zeros16)

    xc, xa, xb = x_cdr3b, x_tra_peptide, x_trb_peptide
    for l in range(L):
        xc4 = xc.reshape(4 * N, 32)
        xa4 = xa.reshape(4 * N, 32)
        xb4 = xb.reshape(4 * N, 32)
        p_b2a, p_b2b = _sc_segsum2(xc4, xc4, (idxs[0], idxs[1]),
                                   (dsts[0], dsts[1]), zeros32)
        p_a2c, p_b2c = _sc_segsum2(xa4, xb4, (idxs[2], idxs[3]),
                                   (dsts[2], dsts[3]), zeros32)
        xa_new = _tc_sage1(cnt[0], p_b2a, xa, Wl_all[l, 0],
                           bl_all[l, 0].reshape(1, 128), Wr_all[l, 0])
        xb_new = _tc_sage1(cnt[1], p_b2b, xb, Wl_all[l, 1],
                           bl_all[l, 1].reshape(1, 128), Wr_all[l, 1])
        xc_new = _tc_sage2(cnt[2], p_a2c, cnt[3], p_b2c, xc,
                           Wl_all[l, 2], Wl_all[l, 3],
                           (bl_all[l, 2] + bl_all[l, 3]).reshape(1, 128),
                           Wr_all[l, 2] + Wr_all[l, 3])
        xc, xa, xb = xc_new, xa_new, xb_new

    g_ca, g_a = _sc_pair_gather2(xc, xa, pidx[0], pidx[1])
    g_cb, g_b = _sc_pair_gather2(xc, xb, pidx[2], pidx[3])

    w3a_p = jnp.zeros((256, 128), f32).at[:, 0:1].set(W3a)
    b3a_p = jnp.zeros((1, 128), f32).at[:, 0:1].set(b3a.reshape(1, 1))
    w3b_p = jnp.zeros((256, 128), f32).at[:, 0:1].set(W3b)
    b3b_p = jnp.zeros((1, 128), f32).at[:, 0:1].set(b3b.reshape(1, 1))

    out_a = _tc_mlp(g_ca, g_a, W1[:128], W1[128:], b1.reshape(1, 512),
                    g1.reshape(1, 512), be1.reshape(1, 512), W2,
                    b2.reshape(1, 256), g2.reshape(1, 256),
                    be2.reshape(1, 256), w3a_p, b3a_p)
    out_b = _tc_mlp(g_cb, g_b, W1[:128], W1[128:], b1.reshape(1, 512),
                    g1.reshape(1, 512), be1.reshape(1, 512), W2,
                    b2.reshape(1, 256), g2.reshape(1, 256),
                    be2.reshape(1, 256), w3b_p, b3b_p)
    return (out_a[:P, 0:1], out_b[:P, 0:1])
